# baseline (speedup 1.0000x reference)
import jax
import jax.numpy as jnp
from jax.experimental import pallas as pl
from jax.experimental.pallas import tpu as pltpu


def _bf16_split3(v):
    h1 = jax.lax.reduce_precision(v, exponent_bits=8, mantissa_bits=7)
    r1 = v - h1
    h2 = jax.lax.reduce_precision(r1, exponent_bits=8, mantissa_bits=7)
    h3 = r1 - h2
    return h1, h2, h3


def _chamfer_block(xa_ref, ya_ref, d1_ref, d2_ref):
    xa = xa_ref[0]
    ya = ya_ref[0]
    d = jax.lax.dot_general(
        xa, ya, (((1,), (1,)), ((), ())), preferred_element_type=jnp.float32
    )
    m = d.shape[1]
    t = d[:, 0:128]
    for k in range(1, m // 128):
        t = jnp.minimum(t, d[:, k * 128:(k + 1) * 128])
    d1_ref[0, 0] = jnp.maximum(jnp.min(t.T, axis=0), 0.0)
    d2_ref[0, 0] = jnp.maximum(jnp.min(d, axis=0), 0.0)


@jax.jit
def _chamfer(xyz1, xyz2):
    b, n, _ = xyz1.shape
    m = xyz2.shape[1]
    nx = jnp.sum(xyz1 * xyz1, axis=2, keepdims=True)
    ny = jnp.sum(xyz2 * xyz2, axis=2, keepdims=True)
    nx1, nx2, nx3 = _bf16_split3(nx)
    ny1, ny2, ny3 = _bf16_split3(ny)
    one_x = jnp.ones_like(nx)
    one_y = jnp.ones_like(ny)
    xa = jnp.concatenate([xyz1, nx1, nx2, nx3, one_x, one_x, one_x], axis=2)
    ya = jnp.concatenate(
        [-2.0 * xyz2, one_y, one_y, one_y, ny1, ny2, ny3], axis=2
    )
    d1, d2 = pl.pallas_call(
        _chamfer_block,
        grid=(b,),
        in_specs=[
            pl.BlockSpec((1, n, 9), lambda bi: (bi, 0, 0)),
            pl.BlockSpec((1, m, 9), lambda bi: (bi, 0, 0)),
        ],
        out_specs=[
            pl.BlockSpec((1, 1, n), lambda bi: (bi, 0, 0)),
            pl.BlockSpec((1, 1, m), lambda bi: (bi, 0, 0)),
        ],
        out_shape=[
            jax.ShapeDtypeStruct((b, 1, n), jnp.float32),
            jax.ShapeDtypeStruct((b, 1, m), jnp.float32),
        ],
        compiler_params=pltpu.CompilerParams(
            dimension_semantics=("arbitrary",),
        ),
    )(xa, ya)
    return d1[:, 0, :], d2[:, 0, :]


def kernel(xyz1, xyz2):
    d1, d2 = _chamfer(xyz1, xyz2)
    return (d1, d2)


# speedup vs baseline: 1.0196x; 1.0196x over previous
import jax
import jax.numpy as jnp
from jax.experimental import pallas as pl
from jax.experimental.pallas import tpu as pltpu


def _bf16_split3(v):
    h1 = jax.lax.reduce_precision(v, exponent_bits=8, mantissa_bits=7)
    r1 = v - h1
    h2 = jax.lax.reduce_precision(r1, exponent_bits=8, mantissa_bits=7)
    h3 = r1 - h2
    return h1, h2, h3


def _chamfer_block(xa_ref, ya_ref, d1_ref, d2_ref):
    b = xa_ref.shape[0]
    for i in range(b):
        xa = xa_ref[i]
        ya = ya_ref[i]
        d = jax.lax.dot_general(
            xa, ya, (((1,), (1,)), ((), ())),
            preferred_element_type=jnp.float32,
        )
        m = d.shape[1]
        t = d[:, 0:128]
        for k in range(1, m // 128):
            t = jnp.minimum(t, d[:, k * 128:(k + 1) * 128])
        d1_ref[i, 0] = jnp.maximum(jnp.min(t.T, axis=0), 0.0)
        d2_ref[i, 0] = jnp.maximum(jnp.min(d, axis=0), 0.0)


@jax.jit
def _chamfer(xyz1, xyz2):
    b, n, _ = xyz1.shape
    m = xyz2.shape[1]
    nx = jnp.sum(xyz1 * xyz1, axis=2, keepdims=True)
    ny = jnp.sum(xyz2 * xyz2, axis=2, keepdims=True)
    nx1, nx2, nx3 = _bf16_split3(nx)
    ny1, ny2, ny3 = _bf16_split3(ny)
    one_x = jnp.ones_like(nx)
    one_y = jnp.ones_like(ny)
    xa = jnp.concatenate(
        [xyz1, nx1, nx2, nx3, one_x, one_x, one_x], axis=2
    ).astype(jnp.bfloat16)
    ya = jnp.concatenate(
        [-2.0 * xyz2, one_y, one_y, one_y, ny1, ny2, ny3], axis=2
    ).astype(jnp.bfloat16)
    d1, d2 = pl.pallas_call(
        _chamfer_block,
        grid=(1,),
        in_specs=[
            pl.BlockSpec((b, n, 9), lambda g: (0, 0, 0)),
            pl.BlockSpec((b, m, 9), lambda g: (0, 0, 0)),
        ],
        out_specs=[
            pl.BlockSpec((b, 1, n), lambda g: (0, 0, 0)),
            pl.BlockSpec((b, 1, m), lambda g: (0, 0, 0)),
        ],
        out_shape=[
            jax.ShapeDtypeStruct((b, 1, n), jnp.float32),
            jax.ShapeDtypeStruct((b, 1, m), jnp.float32),
        ],
        compiler_params=pltpu.CompilerParams(
            dimension_semantics=("arbitrary",),
        ),
    )(xa, ya)
    return d1[:, 0, :], d2[:, 0, :]


def kernel(xyz1, xyz2):
    d1, d2 = _chamfer(xyz1, xyz2)
    return (d1, d2)
